# Initial kernel scaffold; baseline (speedup 1.0000x reference)
import jax
import jax.numpy as jnp
from jax.experimental import pallas as pl


def kernel(embeddings, edge_index, edge_weight, W1, b1, W2, b2):
    raise NotImplementedError("write your pallas kernel here")


# speedup vs baseline: 3.2584x; 3.2584x over previous
import functools

import jax
import jax.numpy as jnp
from jax import lax
from jax.experimental import pallas as pl
from jax.experimental.pallas import tpu as pltpu
from jax.experimental.pallas import tpu_sc as plsc

N = 10000
E = 320000
D = 128

NC = 2
NS = 16
PAD_N = 10240
HALF = PAD_N // NC
TRASH = HALF
STRIPE = E // NS
CHUNK = 80
TRIPS = STRIPE // CHUNK
ROWS_PER_TILE = HALF // NS
ZROWS = 64


def _spmm_sc_body(x_hbm, src_hbm, dst_hbm, w_hbm, b_hbm, out_hbm,
                  src_v, dst_v, w_v, rows_v, init_v, bias_v, dstst_v,
                  accum, sem):
    c = lax.axis_index("c")
    s = lax.axis_index("s")
    lo = c * HALF

    pltpu.sync_copy(b_hbm, bias_v)

    def irow(e, _):
        for q in range(D // 16):
            init_v[e, pl.ds(q * 16, 16)] = bias_v[pl.ds(q * 16, 16)]
        return 0
    lax.fori_loop(0, ZROWS, irow, 0)
    row0 = s * ROWS_PER_TILE
    for k in range(ROWS_PER_TILE // ZROWS):
        pltpu.sync_copy(init_v, accum.at[pl.ds(row0 + k * ZROWS, ZROWS)])

    pltpu.sync_copy(src_hbm.at[s], src_v)
    pltpu.sync_copy(dst_hbm.at[s], dst_v)
    pltpu.sync_copy(w_hbm.at[s], w_v)

    plsc.subcore_barrier()

    lo_v = jnp.broadcast_to(lo, (16,))
    half_v = jnp.full((16,), HALF, jnp.int32)
    trash_v = jnp.full((16,), TRASH, jnp.int32)
    zero_v = jnp.zeros((16,), jnp.int32)

    def chunk_body(j, _):
        base = j * CHUNK
        pltpu.async_copy(x_hbm.at[src_v.at[pl.ds(base, CHUNK)]],
                         rows_v, sem).wait()

        for g in range(CHUNK // 16):
            w16 = w_v[pl.ds(base + g * 16, 16)]
            rel = dst_v[pl.ds(base + g * 16, 16)] - lo_v
            rel = jnp.where(rel >= zero_v,
                            jnp.where(rel < half_v, rel, trash_v), trash_v)
            dstst_v[0, pl.ds(g * 16, 16)] = rel
            for l in range(16):
                wvec = jnp.broadcast_to(w16[l], (16,))
                e = g * 16 + l
                for q in range(D // 16):
                    rows_v[e, pl.ds(q * 16, 16)] = (
                        rows_v[e, pl.ds(q * 16, 16)] * wvec)

        pltpu.sync_copy(rows_v, accum.at[dstst_v.at[0]], add=True)
        return 0
    lax.fori_loop(0, TRIPS, chunk_body, 0)

    plsc.subcore_barrier()
    pltpu.sync_copy(accum.at[pl.ds(row0, ROWS_PER_TILE)],
                    out_hbm.at[pl.ds(lo + row0, ROWS_PER_TILE)])


_spmm_sc = functools.partial(
    pl.kernel,
    out_type=jax.ShapeDtypeStruct((PAD_N, D), jnp.float32),
    mesh=plsc.VectorSubcoreMesh(core_axis_name="c", subcore_axis_name="s"),
    scratch_types=[
        pltpu.VMEM((STRIPE,), jnp.int32),
        pltpu.VMEM((STRIPE,), jnp.int32),
        pltpu.VMEM((STRIPE,), jnp.float32),
        pltpu.VMEM((CHUNK, D), jnp.float32),
        pltpu.VMEM((ZROWS, D), jnp.float32),
        pltpu.VMEM((D,), jnp.float32),
        pltpu.VMEM((1, CHUNK), jnp.int32),
        pltpu.VMEM_SHARED((HALF + 8, D), jnp.float32),
        pltpu.SemaphoreType.DMA,
    ],
)(_spmm_sc_body)


def _mm_body(x_ref, w_ref, o_ref):
    o_ref[...] = jnp.dot(x_ref[...], w_ref[...],
                         preferred_element_type=jnp.float32)


def _relu_mm_body(x_ref, w_ref, o_ref):
    o_ref[...] = jnp.dot(jax.nn.relu(x_ref[...]), w_ref[...],
                         preferred_element_type=jnp.float32)


def _mm(x, w, body):
    blk = N // 10
    return pl.pallas_call(
        body,
        grid=(10,),
        in_specs=[
            pl.BlockSpec((blk, D), lambda i: (i, 0)),
            pl.BlockSpec((D, D), lambda i: (0, 0)),
        ],
        out_specs=pl.BlockSpec((blk, D), lambda i: (i, 0)),
        out_shape=jax.ShapeDtypeStruct((N, D), jnp.float32),
    )(x, w)


def kernel(embeddings, edge_index, edge_weight, W1, b1, W2, b2):
    src = edge_index[0].reshape(NS, STRIPE)
    dst = edge_index[1].reshape(NS, STRIPE)
    ww = edge_weight.reshape(NS, STRIPE)

    s1 = _mm(embeddings, W1, _mm_body)
    h1 = _spmm_sc(s1, src, dst, ww, b1)
    s2 = _mm(h1, W2, _relu_mm_body)
    out = _spmm_sc(s2, src, dst, ww, b2)
    return out[:N]


# speedup vs baseline: 5.3024x; 1.6273x over previous
import functools

import jax
import jax.numpy as jnp
from jax import lax
from jax.experimental import pallas as pl
from jax.experimental.pallas import tpu as pltpu
from jax.experimental.pallas import tpu_sc as plsc

N = 10000
E = 320000
D = 128

NC = 2
NS = 16
PAD_N = 10240
HALF = PAD_N // NC
TRASH = HALF
STRIPE = E // NS
CHUNK = 80
TRIPS = STRIPE // CHUNK
ROWS_PER_TILE = HALF // NS
ZROWS = 64


def _spmm_sc_body(x_hbm, src_hbm, dst_hbm, w_hbm, b_hbm, out_hbm,
                  src_v, dst_v, w_v, rows_v, rows2_v, init_v, bias_v,
                  dstst_v, accum, sem, sem2):
    c = lax.axis_index("c")
    s = lax.axis_index("s")
    lo = c * HALF

    pltpu.sync_copy(b_hbm, bias_v)

    def irow(e, _):
        for q in range(D // 16):
            init_v[e, pl.ds(q * 16, 16)] = bias_v[pl.ds(q * 16, 16)]
        return 0
    lax.fori_loop(0, ZROWS, irow, 0)
    row0 = s * ROWS_PER_TILE
    for k in range(ROWS_PER_TILE // ZROWS):
        pltpu.sync_copy(init_v, accum.at[pl.ds(row0 + k * ZROWS, ZROWS)])

    pltpu.sync_copy(src_hbm.at[s], src_v)
    pltpu.sync_copy(dst_hbm.at[s], dst_v)
    pltpu.sync_copy(w_hbm.at[s], w_v)

    plsc.subcore_barrier()

    lo_v = jnp.broadcast_to(lo, (16,))
    half_v = jnp.full((16,), HALF, jnp.int32)
    trash_v = jnp.full((16,), TRASH, jnp.int32)
    zero_v = jnp.zeros((16,), jnp.int32)

    def start_gather(j, buf, gsem):
        pltpu.async_copy(x_hbm.at[src_v.at[pl.ds(j * CHUNK, CHUNK)]],
                         buf, gsem)

    def wait_gather(buf, gsem):
        pltpu.make_async_copy(x_hbm.at[pl.ds(0, CHUNK)], buf, gsem).wait()

    def process(j, buf):
        base = j * CHUNK
        for g in range(CHUNK // 16):
            w16 = w_v[pl.ds(base + g * 16, 16)]
            rel = dst_v[pl.ds(base + g * 16, 16)] - lo_v
            rel = jnp.where(rel >= zero_v,
                            jnp.where(rel < half_v, rel, trash_v), trash_v)
            dstst_v[0, pl.ds(g * 16, 16)] = rel
            for l in range(16):
                wvec = jnp.broadcast_to(w16[l], (16,))
                e = g * 16 + l
                for q in range(D // 16):
                    buf[e, pl.ds(q * 16, 16)] = (
                        buf[e, pl.ds(q * 16, 16)] * wvec)
        pltpu.sync_copy(buf, accum.at[dstst_v.at[0]], add=True)

    start_gather(0, rows_v, sem)
    start_gather(1, rows2_v, sem2)

    def pair_body(k, _):
        j0 = 2 * k
        wait_gather(rows_v, sem)
        process(j0, rows_v)

        @pl.when(j0 + 2 < TRIPS)
        def _():
            start_gather(j0 + 2, rows_v, sem)
        wait_gather(rows2_v, sem2)
        process(j0 + 1, rows2_v)

        @pl.when(j0 + 3 < TRIPS)
        def _():
            start_gather(j0 + 3, rows2_v, sem2)
        return 0
    lax.fori_loop(0, TRIPS // 2, pair_body, 0)

    plsc.subcore_barrier()
    pltpu.sync_copy(accum.at[pl.ds(row0, ROWS_PER_TILE)],
                    out_hbm.at[pl.ds(lo + row0, ROWS_PER_TILE)])


_spmm_sc = functools.partial(
    pl.kernel,
    out_type=jax.ShapeDtypeStruct((PAD_N, D), jnp.float32),
    mesh=plsc.VectorSubcoreMesh(core_axis_name="c", subcore_axis_name="s"),
    scratch_types=[
        pltpu.VMEM((STRIPE,), jnp.int32),
        pltpu.VMEM((STRIPE,), jnp.int32),
        pltpu.VMEM((STRIPE,), jnp.float32),
        pltpu.VMEM((CHUNK, D), jnp.float32),
        pltpu.VMEM((CHUNK, D), jnp.float32),
        pltpu.VMEM((ZROWS, D), jnp.float32),
        pltpu.VMEM((D,), jnp.float32),
        pltpu.VMEM((1, CHUNK), jnp.int32),
        pltpu.VMEM_SHARED((HALF + 8, D), jnp.float32),
        pltpu.SemaphoreType.DMA,
        pltpu.SemaphoreType.DMA,
    ],
)(_spmm_sc_body)


def _mm_body(x_ref, w_ref, o_ref):
    o_ref[...] = jnp.dot(x_ref[...], w_ref[...],
                         preferred_element_type=jnp.float32)


def _relu_mm_body(x_ref, w_ref, o_ref):
    o_ref[...] = jnp.dot(jax.nn.relu(x_ref[...]), w_ref[...],
                         preferred_element_type=jnp.float32)


def _mm(x, w, body):
    blk = N // 10
    return pl.pallas_call(
        body,
        grid=(10,),
        in_specs=[
            pl.BlockSpec((blk, D), lambda i: (i, 0)),
            pl.BlockSpec((D, D), lambda i: (0, 0)),
        ],
        out_specs=pl.BlockSpec((blk, D), lambda i: (i, 0)),
        out_shape=jax.ShapeDtypeStruct((N, D), jnp.float32),
    )(x, w)


def kernel(embeddings, edge_index, edge_weight, W1, b1, W2, b2):
    src = edge_index[0].reshape(NS, STRIPE)
    dst = edge_index[1].reshape(NS, STRIPE)
    ww = edge_weight.reshape(NS, STRIPE)

    s1 = _mm(embeddings, W1, _mm_body)
    h1 = _spmm_sc(s1, src, dst, ww, b1)
    s2 = _mm(h1, W2, _relu_mm_body)
    out = _spmm_sc(s2, src, dst, ww, b2)
    return out[:N]
